# baseline (speedup 1.0000x reference)
import functools

import jax
import jax.numpy as jnp
from jax import lax
from jax.experimental import pallas as pl
from jax.experimental.pallas import tpu as pltpu
from jax.experimental.pallas import tpu_sc as plsc

L = 16
NW = 32
CH = 3200
GROUPS = CH // L
N_BINS = 15
C15 = 1.0 / 15.0
KPACK = 4096.0


def _ece_body(nchunks, lt_hbm, lab_hbm, out_hbm,
              pa, lba, pb, lbb, ca_v, cf_v, sem0, sem1):
    cid = lax.axis_index("c")
    sid = lax.axis_index("s")
    wid = sid * 2 + cid

    zeros = jnp.zeros((L,), jnp.float32)
    for i in range(N_BINS):
        ca_v[pl.ds(i * L, L)] = zeros
        cf_v[pl.ds(i * L, L)] = zeros

    lane = lax.broadcasted_iota(jnp.int32, (L,), 0)
    row0 = jnp.zeros((L,), jnp.int32)
    row1 = jnp.full((L,), 1, jnp.int32)
    row2 = jnp.full((L,), 2, jnp.int32)
    bufs = ((pa, lba), (pb, lbb))
    sems = (sem0, sem1)

    def _copies(j, b):
        c = wid + NW * j
        off = pl.multiple_of(c * CH, 128)
        pv, lbv = bufs[b]
        return c, [
            pltpu.make_async_copy(
                lt_hbm.at[:, pl.ds(off, CH)], pv, sems[b]),
            pltpu.make_async_copy(lab_hbm.at[pl.ds(off, CH)], lbv, sems[b]),
        ]

    def start(j, b):
        c, copies = _copies(j, b)

        @pl.when(c < nchunks)
        def _():
            for cp in copies:
                cp.start()

    def wait(j, b):
        c, copies = _copies(j, b)

        @pl.when(c < nchunks)
        def _():
            for cp in copies:
                cp.wait()

    def compute(j, b):
        c = wid + NW * j
        pv, lb_v = bufs[b]

        @pl.when(c < nchunks)
        def _():
            @plsc.parallel_loop(0, CH, step=L, unroll=8)
            def grp(base):
                col = base + lane
                l0 = plsc.load_gather(pv, [row0, col])
                l1 = plsc.load_gather(pv, [row1, col])
                l2 = plsc.load_gather(pv, [row2, col])
                lb = lb_v[pl.ds(base, L)]

                m01 = jnp.maximum(l0, l1)
                lmax = jnp.maximum(m01, l2)
                s = (jnp.exp(l0 - lmax) + jnp.exp(l1 - lmax)
                     + jnp.exp(l2 - lmax))
                conf = 1.0 / s
                pred = jnp.where(l1 > l0, 1, 0).astype(jnp.int32)
                pred = jnp.where(l2 > m01, 2, pred)
                accf = jnp.where(pred == lb, 1.0, 0.0).astype(jnp.float32)

                b0 = (conf * 15.0).astype(jnp.int32)
                b0f = b0.astype(jnp.float32)
                lo = b0f * C15
                hi = (b0f + 1.0) * C15
                bb = b0 - jnp.where(conf <= lo, 1, 0) \
                    + jnp.where(conf > hi, 1, 0)

                slot = bb * L + lane
                plsc.addupdate_scatter(ca_v, [slot], accf + KPACK)
                plsc.addupdate_scatter(cf_v, [slot], conf)

    niter = (nchunks + NW - 1) // NW
    niter2 = (niter + 2) // 2

    start(0, 0)
    start(1, 1)

    def outer(j2, _):
        for b in (0, 1):
            j = 2 * j2 + b
            wait(j, b)
            compute(j, b)
            start(j + 2, b)
        return 0

    lax.fori_loop(0, niter2, outer, 0)
    pltpu.sync_copy(ca_v, out_hbm.at[2 * wid])
    pltpu.sync_copy(cf_v, out_hbm.at[2 * wid + 1])


def kernel(logits, labels):
    n = logits.shape[0]
    assert n % CH == 0
    nchunks = n // CH

    lt = logits.T

    mesh = plsc.VectorSubcoreMesh(
        core_axis_name="c", subcore_axis_name="s", num_cores=2, num_subcores=16
    )
    run = pl.kernel(
        functools.partial(_ece_body, nchunks),
        out_type=jax.ShapeDtypeStruct((NW * 2, N_BINS * L), jnp.float32),
        mesh=mesh,
        compiler_params=pltpu.CompilerParams(needs_layout_passes=False),
        scratch_types=[
            pltpu.VMEM((3, CH), jnp.float32),
            pltpu.VMEM((CH,), jnp.int32),
            pltpu.VMEM((3, CH), jnp.float32),
            pltpu.VMEM((CH,), jnp.int32),
            pltpu.VMEM((N_BINS * L,), jnp.float32),
            pltpu.VMEM((N_BINS * L,), jnp.float32),
            pltpu.SemaphoreType.DMA,
            pltpu.SemaphoreType.DMA,
        ],
    )
    assert ((nchunks + NW - 1) // NW) * (CH // L) * (KPACK + 1) < 2 ** 24

    parts = run(lt, labels)

    r = parts.reshape(NW, 2, N_BINS, L)
    ca_p = r[:, 0]
    cnt_p = jnp.floor(ca_p / KPACK)
    cnt = cnt_p.sum(axis=(0, 2))
    sacc = (ca_p - cnt_p * KPACK).sum(axis=(0, 2))
    sconf = r[:, 1].sum(axis=(0, 2))
    n_total = jnp.asarray(n, dtype=jnp.float32)
    prop = cnt / n_total
    safe = jnp.maximum(cnt, 1.0)
    contrib = jnp.abs(sconf / safe - sacc / safe) * prop
    return jnp.sum(jnp.where(prop > 0.0, contrib, 0.0)).astype(jnp.float32)


# speedup vs baseline: 1.1021x; 1.1021x over previous
import functools

import jax
import jax.numpy as jnp
from jax import lax
from jax.experimental import pallas as pl
from jax.experimental.pallas import tpu as pltpu
from jax.experimental.pallas import tpu_sc as plsc

L = 16
NW = 32
CH = 3200
GROUPS = CH // L
N_BINS = 15
C15 = 1.0 / 15.0
KPACK = 4096.0


def _ece_body(nchunks, lt_hbm, lab_hbm, out_hbm,
              pa, lba, pb, lbb, ca_v, cf_v, sem0, sem1):
    cid = lax.axis_index("c")
    sid = lax.axis_index("s")
    wid = sid * 2 + cid

    zeros = jnp.zeros((L,), jnp.float32)
    for i in range(N_BINS):
        ca_v[pl.ds(i * L, L)] = zeros
        cf_v[pl.ds(i * L, L)] = zeros

    lane = lax.broadcasted_iota(jnp.int32, (L,), 0)
    row0 = jnp.zeros((L,), jnp.int32)
    row1 = jnp.full((L,), 1, jnp.int32)
    row2 = jnp.full((L,), 2, jnp.int32)
    bufs = ((pa, lba), (pb, lbb))
    sems = (sem0, sem1)

    def _copies(j, b):
        c = wid + NW * j
        off = pl.multiple_of(c * CH, 128)
        pv, lbv = bufs[b]
        return c, [
            pltpu.make_async_copy(
                lt_hbm.at[:, pl.ds(off, CH)], pv, sems[b]),
            pltpu.make_async_copy(lab_hbm.at[pl.ds(off, CH)], lbv, sems[b]),
        ]

    def start(j, b):
        c, copies = _copies(j, b)

        @pl.when(c < nchunks)
        def _():
            for cp in copies:
                cp.start()

    def wait(j, b):
        c, copies = _copies(j, b)

        @pl.when(c < nchunks)
        def _():
            for cp in copies:
                cp.wait()

    def compute(j, b):
        c = wid + NW * j
        pv, lb_v = bufs[b]

        @pl.when(c < nchunks)
        def _():
            @plsc.parallel_loop(0, CH, step=L, unroll=4)
            def grp(base):
                col = base + lane
                l0 = plsc.load_gather(pv, [row0, col])
                l1 = plsc.load_gather(pv, [row1, col])
                l2 = plsc.load_gather(pv, [row2, col])
                lb = lb_v[pl.ds(base, L)]

                m01 = jnp.maximum(l0, l1)
                lmax = jnp.maximum(m01, l2)
                s = (jnp.exp(l0 - lmax) + jnp.exp(l1 - lmax)
                     + jnp.exp(l2 - lmax))
                conf = 1.0 / s
                pred = jnp.where(l1 > l0, 1, 0).astype(jnp.int32)
                pred = jnp.where(l2 > m01, 2, pred)
                accf = jnp.where(pred == lb, 1.0, 0.0).astype(jnp.float32)

                b0 = (conf * 15.0).astype(jnp.int32)
                b0f = b0.astype(jnp.float32)
                lo = b0f * C15
                hi = (b0f + 1.0) * C15
                bb = b0 - jnp.where(conf <= lo, 1, 0) \
                    + jnp.where(conf > hi, 1, 0)

                slot = bb * L + lane
                plsc.addupdate_scatter(ca_v, [slot], accf + KPACK)
                plsc.addupdate_scatter(cf_v, [slot], conf)

    niter = (nchunks + NW - 1) // NW
    niter2 = (niter + 2) // 2

    start(0, 0)
    start(1, 1)

    def outer(j2, _):
        for b in (0, 1):
            j = 2 * j2 + b
            wait(j, b)
            compute(j, b)
            start(j + 2, b)
        return 0

    lax.fori_loop(0, niter2, outer, 0)
    pltpu.sync_copy(ca_v, out_hbm.at[2 * wid])
    pltpu.sync_copy(cf_v, out_hbm.at[2 * wid + 1])


def kernel(logits, labels):
    n = logits.shape[0]
    assert n % CH == 0
    nchunks = n // CH

    lt = logits.T

    mesh = plsc.VectorSubcoreMesh(
        core_axis_name="c", subcore_axis_name="s", num_cores=2, num_subcores=16
    )
    run = pl.kernel(
        functools.partial(_ece_body, nchunks),
        out_type=jax.ShapeDtypeStruct((NW * 2, N_BINS * L), jnp.float32),
        mesh=mesh,
        compiler_params=pltpu.CompilerParams(needs_layout_passes=False),
        scratch_types=[
            pltpu.VMEM((3, CH), jnp.float32),
            pltpu.VMEM((CH,), jnp.int32),
            pltpu.VMEM((3, CH), jnp.float32),
            pltpu.VMEM((CH,), jnp.int32),
            pltpu.VMEM((N_BINS * L,), jnp.float32),
            pltpu.VMEM((N_BINS * L,), jnp.float32),
            pltpu.SemaphoreType.DMA,
            pltpu.SemaphoreType.DMA,
        ],
    )
    assert ((nchunks + NW - 1) // NW) * (CH // L) * (KPACK + 1) < 2 ** 24

    parts = run(lt, labels)

    r = parts.reshape(NW, 2, N_BINS, L)
    ca_p = r[:, 0]
    cnt_p = jnp.floor(ca_p / KPACK)
    cnt = cnt_p.sum(axis=(0, 2))
    sacc = (ca_p - cnt_p * KPACK).sum(axis=(0, 2))
    sconf = r[:, 1].sum(axis=(0, 2))
    n_total = jnp.asarray(n, dtype=jnp.float32)
    prop = cnt / n_total
    safe = jnp.maximum(cnt, 1.0)
    contrib = jnp.abs(sconf / safe - sacc / safe) * prop
    return jnp.sum(jnp.where(prop > 0.0, contrib, 0.0)).astype(jnp.float32)


# speedup vs baseline: 1.2327x; 1.1185x over previous
import functools

import jax
import jax.numpy as jnp
from jax import lax
from jax.experimental import pallas as pl
from jax.experimental.pallas import tpu as pltpu
from jax.experimental.pallas import tpu_sc as plsc

L = 16
NW = 32
CH = 3200
GROUPS = CH // L
N_BINS = 15
C15 = 1.0 / 15.0
KPACK = 4096.0


def _ece_body(nchunks, lt_hbm, lab_hbm, out_hbm,
              pa, lba, pb, lbb, ca_v, cf_v, sem0, sem1):
    cid = lax.axis_index("c")
    sid = lax.axis_index("s")
    wid = sid * 2 + cid

    zeros = jnp.zeros((L,), jnp.float32)
    for i in range(N_BINS):
        ca_v[pl.ds(i * L, L)] = zeros
        cf_v[pl.ds(i * L, L)] = zeros

    lane = lax.broadcasted_iota(jnp.int32, (L,), 0)
    row0 = jnp.zeros((L,), jnp.int32)
    row1 = jnp.full((L,), 1, jnp.int32)
    row2 = jnp.full((L,), 2, jnp.int32)
    bufs = ((pa, lba), (pb, lbb))
    sems = (sem0, sem1)

    def _copies(j, b):
        c = wid + NW * j
        off = pl.multiple_of(c * CH, 128)
        pv, lbv = bufs[b]
        return c, [
            pltpu.make_async_copy(
                lt_hbm.at[:, pl.ds(off, CH)], pv, sems[b]),
            pltpu.make_async_copy(lab_hbm.at[pl.ds(off, CH)], lbv, sems[b]),
        ]

    def start(j, b):
        c, copies = _copies(j, b)

        @pl.when(c < nchunks)
        def _():
            for cp in copies:
                cp.start()

    def wait(j, b):
        c, copies = _copies(j, b)

        @pl.when(c < nchunks)
        def _():
            for cp in copies:
                cp.wait()

    def compute(j, b):
        c = wid + NW * j
        pv, lb_v = bufs[b]

        @pl.when(c < nchunks)
        def _():
            @plsc.parallel_loop(0, CH, step=L, unroll=4)
            def grp(base):
                l0 = pv[0, pl.ds(base, L)]
                l1 = pv[1, pl.ds(base, L)]
                l2 = pv[2, pl.ds(base, L)]
                lb = lb_v[pl.ds(base, L)]

                m01 = jnp.maximum(l0, l1)
                lmax = jnp.maximum(m01, l2)
                s = (jnp.exp(l0 - lmax) + jnp.exp(l1 - lmax)
                     + jnp.exp(l2 - lmax))
                conf = 1.0 / s
                pred = jnp.where(l1 > l0, 1, 0).astype(jnp.int32)
                pred = jnp.where(l2 > m01, 2, pred)
                accf = jnp.where(pred == lb, 1.0, 0.0).astype(jnp.float32)

                b0 = (conf * 15.0).astype(jnp.int32)
                b0f = b0.astype(jnp.float32)
                lo = b0f * C15
                hi = (b0f + 1.0) * C15
                bb = b0 - jnp.where(conf <= lo, 1, 0) \
                    + jnp.where(conf > hi, 1, 0)

                slot = bb * L + lane
                plsc.addupdate_scatter(ca_v, [slot], accf + KPACK)
                plsc.addupdate_scatter(cf_v, [slot], conf)

    niter = (nchunks + NW - 1) // NW
    niter2 = (niter + 2) // 2

    start(0, 0)
    start(1, 1)

    def outer(j2, _):
        for b in (0, 1):
            j = 2 * j2 + b
            wait(j, b)
            compute(j, b)
            start(j + 2, b)
        return 0

    lax.fori_loop(0, niter2, outer, 0)
    pltpu.sync_copy(ca_v, out_hbm.at[2 * wid])
    pltpu.sync_copy(cf_v, out_hbm.at[2 * wid + 1])


def kernel(logits, labels):
    n = logits.shape[0]
    assert n % CH == 0
    nchunks = n // CH

    lt = logits.T

    mesh = plsc.VectorSubcoreMesh(
        core_axis_name="c", subcore_axis_name="s", num_cores=2, num_subcores=16
    )
    run = pl.kernel(
        functools.partial(_ece_body, nchunks),
        out_type=jax.ShapeDtypeStruct((NW * 2, N_BINS * L), jnp.float32),
        mesh=mesh,
        compiler_params=pltpu.CompilerParams(needs_layout_passes=False),
        scratch_types=[
            pltpu.VMEM((3, CH), jnp.float32),
            pltpu.VMEM((CH,), jnp.int32),
            pltpu.VMEM((3, CH), jnp.float32),
            pltpu.VMEM((CH,), jnp.int32),
            pltpu.VMEM((N_BINS * L,), jnp.float32),
            pltpu.VMEM((N_BINS * L,), jnp.float32),
            pltpu.SemaphoreType.DMA,
            pltpu.SemaphoreType.DMA,
        ],
    )
    assert ((nchunks + NW - 1) // NW) * (CH // L) * (KPACK + 1) < 2 ** 24

    parts = run(lt, labels)

    r = parts.reshape(NW, 2, N_BINS, L)
    ca_p = r[:, 0]
    cnt_p = jnp.floor(ca_p / KPACK)
    cnt = cnt_p.sum(axis=(0, 2))
    sacc = (ca_p - cnt_p * KPACK).sum(axis=(0, 2))
    sconf = r[:, 1].sum(axis=(0, 2))
    n_total = jnp.asarray(n, dtype=jnp.float32)
    prop = cnt / n_total
    safe = jnp.maximum(cnt, 1.0)
    contrib = jnp.abs(sconf / safe - sacc / safe) * prop
    return jnp.sum(jnp.where(prop > 0.0, contrib, 0.0)).astype(jnp.float32)


# speedup vs baseline: 1.3388x; 1.0861x over previous
import functools

import jax
import jax.numpy as jnp
from jax import lax
from jax.experimental import pallas as pl
from jax.experimental.pallas import tpu as pltpu
from jax.experimental.pallas import tpu_sc as plsc

L = 16
NW = 32
CH = 3200
GROUPS = CH // L
N_BINS = 15
C15 = 1.0 / 15.0
KPACK = 4096.0


def _ece_body(nchunks, lt_hbm, lab_hbm, out_hbm,
              pa, lba, pb, lbb, ca_v, cf_v, sem0, sem1):
    cid = lax.axis_index("c")
    sid = lax.axis_index("s")
    wid = sid * 2 + cid

    zeros = jnp.zeros((L,), jnp.float32)
    for i in range(N_BINS):
        ca_v[pl.ds(i * L, L)] = zeros
        cf_v[pl.ds(i * L, L)] = zeros

    lane = lax.broadcasted_iota(jnp.int32, (L,), 0)
    row0 = jnp.zeros((L,), jnp.int32)
    row1 = jnp.full((L,), 1, jnp.int32)
    row2 = jnp.full((L,), 2, jnp.int32)
    bufs = ((pa, lba), (pb, lbb))
    sems = (sem0, sem1)

    def _copies(j, b):
        c = wid + NW * j
        off = pl.multiple_of(c * CH, 128)
        pv, lbv = bufs[b]
        return c, [
            pltpu.make_async_copy(
                lt_hbm.at[:, pl.ds(off, CH)], pv, sems[b]),
            pltpu.make_async_copy(lab_hbm.at[pl.ds(off, CH)], lbv, sems[b]),
        ]

    def start(j, b):
        c, copies = _copies(j, b)

        @pl.when(c < nchunks)
        def _():
            for cp in copies:
                cp.start()

    def wait(j, b):
        c, copies = _copies(j, b)

        @pl.when(c < nchunks)
        def _():
            for cp in copies:
                cp.wait()

    def compute(j, b):
        c = wid + NW * j
        pv, lb_v = bufs[b]

        @pl.when(c < nchunks)
        def _():
            @plsc.parallel_loop(0, CH, step=L, unroll=4)
            def grp(base):
                l0 = pv[0, pl.ds(base, L)]
                l1 = pv[1, pl.ds(base, L)]
                l2 = pv[2, pl.ds(base, L)]
                lb = lb_v[pl.ds(base, L)]

                m01 = jnp.maximum(l0, l1)
                lmax = jnp.maximum(m01, l2)
                s = (jnp.exp(l0 - lmax) + jnp.exp(l1 - lmax)
                     + jnp.exp(l2 - lmax))
                conf = 1.0 / s
                pred = jnp.where(l1 > l0, 1, 0).astype(jnp.int32)
                pred = jnp.where(l2 > m01, 2, pred)
                accf = jnp.where(pred == lb, 1.0, 0.0).astype(jnp.float32)

                t = conf * 15.0
                b0 = t.astype(jnp.int32)
                bb = b0 - jnp.where(t == b0.astype(jnp.float32), 1, 0)

                slot = bb * L + lane
                plsc.addupdate_scatter(ca_v, [slot], accf + KPACK)
                plsc.addupdate_scatter(cf_v, [slot], conf)

    niter = (nchunks + NW - 1) // NW
    niter2 = (niter + 2) // 2

    start(0, 0)
    start(1, 1)

    def outer(j2, _):
        for b in (0, 1):
            j = 2 * j2 + b
            wait(j, b)
            compute(j, b)
            start(j + 2, b)
        return 0

    lax.fori_loop(0, niter2, outer, 0)
    pltpu.sync_copy(ca_v, out_hbm.at[2 * wid])
    pltpu.sync_copy(cf_v, out_hbm.at[2 * wid + 1])


def kernel(logits, labels):
    n = logits.shape[0]
    assert n % CH == 0
    nchunks = n // CH

    lt = logits.T

    mesh = plsc.VectorSubcoreMesh(
        core_axis_name="c", subcore_axis_name="s", num_cores=2, num_subcores=16
    )
    run = pl.kernel(
        functools.partial(_ece_body, nchunks),
        out_type=jax.ShapeDtypeStruct((NW * 2, N_BINS * L), jnp.float32),
        mesh=mesh,
        compiler_params=pltpu.CompilerParams(needs_layout_passes=False),
        scratch_types=[
            pltpu.VMEM((3, CH), jnp.float32),
            pltpu.VMEM((CH,), jnp.int32),
            pltpu.VMEM((3, CH), jnp.float32),
            pltpu.VMEM((CH,), jnp.int32),
            pltpu.VMEM((N_BINS * L,), jnp.float32),
            pltpu.VMEM((N_BINS * L,), jnp.float32),
            pltpu.SemaphoreType.DMA,
            pltpu.SemaphoreType.DMA,
        ],
    )
    assert ((nchunks + NW - 1) // NW) * (CH // L) * (KPACK + 1) < 2 ** 24

    parts = run(lt, labels)

    r = parts.reshape(NW, 2, N_BINS, L)
    ca_p = r[:, 0]
    cnt_p = jnp.floor(ca_p / KPACK)
    cnt = cnt_p.sum(axis=(0, 2))
    sacc = (ca_p - cnt_p * KPACK).sum(axis=(0, 2))
    sconf = r[:, 1].sum(axis=(0, 2))
    n_total = jnp.asarray(n, dtype=jnp.float32)
    prop = cnt / n_total
    safe = jnp.maximum(cnt, 1.0)
    contrib = jnp.abs(sconf / safe - sacc / safe) * prop
    return jnp.sum(jnp.where(prop > 0.0, contrib, 0.0)).astype(jnp.float32)
